# baseline (speedup 1.0000x reference)
import functools

import jax
import jax.numpy as jnp
from jax import lax
from jax.experimental import pallas as pl
from jax.experimental.pallas import tpu as pltpu

_NEG = -1e30


def _round_up(x, m):
    return ((x + m - 1) // m) * m


def _gather_ce_kernel(idx_sref, tgt_sref, emb_ref, logits_ref, per_ex_ref,
                      *, tile, unroll, v_pad):
    base = pl.program_id(0) * tile
    lane = lax.broadcasted_iota(jnp.int32, (1, v_pad), 1)[0]

    def outer(o, carry):
        mi0 = o * unroll
        for u in range(unroll):
            mi = mi0 + u
            r = idx_sref[base + mi]
            t = tgt_sref[base + mi]
            row = emb_ref[r, 0, :]
            logits_ref[mi, 0, :] = row
            sel = jnp.where(lane == t, row, 0.0)
            per_ex_ref[mi, 0, :] = -jnp.sum(sel, keepdims=True)
        return carry

    lax.fori_loop(0, tile // unroll, outer, 0)

    x = logits_ref[...]
    m = jnp.max(x, axis=2, keepdims=True)
    s = jnp.sum(jnp.exp(x - m), axis=2, keepdims=True)
    per_ex_ref[...] = m + jnp.log(s) + per_ex_ref[...]


def _pad_1d(tok, bt, bt_pad):
    tok = tok.reshape(bt).astype(jnp.int32)
    if bt_pad != bt:
        tok = jnp.concatenate([tok, jnp.zeros((bt_pad - bt,), jnp.int32)])
    return tok


def kernel(emb_table, idx, targets, *, tile=256, unroll=8):
    B, T = idx.shape
    V = emb_table.shape[0]
    BT = B * T

    v_pad = _round_up(V, 128)
    tile = min(tile, _round_up(BT, 8))
    bt_pad = _round_up(BT, tile)
    num_tiles = bt_pad // tile

    if v_pad == V:
        emb_pad = emb_table.astype(jnp.float32)
    else:
        emb_pad = jnp.full((v_pad, v_pad), _NEG, dtype=jnp.float32)
        emb_pad = emb_pad.at[:V, :V].set(emb_table.astype(jnp.float32))
    emb3 = emb_pad.reshape(v_pad, 1, v_pad)

    idx_flat = _pad_1d(idx, BT, bt_pad)
    tgt_flat = _pad_1d(targets if targets is not None else idx, BT, bt_pad)

    body = functools.partial(_gather_ce_kernel, tile=tile, unroll=unroll,
                             v_pad=v_pad)

    logits3, per_ex = pl.pallas_call(
        body,
        out_shape=(
            jax.ShapeDtypeStruct((bt_pad, 1, v_pad), jnp.float32),
            jax.ShapeDtypeStruct((bt_pad, 1, 1), jnp.float32),
        ),
        grid=(num_tiles,),
        in_specs=[
            pl.BlockSpec(memory_space=pltpu.SMEM),
            pl.BlockSpec(memory_space=pltpu.SMEM),
            pl.BlockSpec((v_pad, 1, v_pad), lambda i: (0, 0, 0)),
        ],
        out_specs=(
            pl.BlockSpec((tile, 1, v_pad), lambda i: (i, 0, 0)),
            pl.BlockSpec((tile, 1, 1), lambda i: (i, 0, 0)),
        ),
        compiler_params=pltpu.CompilerParams(
            dimension_semantics=("parallel",),
            vmem_limit_bytes=48 * 1024 * 1024,
        ),
        cost_estimate=pl.CostEstimate(
            flops=8 * bt_pad * v_pad,
            transcendentals=bt_pad * v_pad,
            bytes_accessed=(v_pad * v_pad * 4 + bt_pad * v_pad * 4
                            + 3 * bt_pad * 4),
        ),
    )(idx_flat, tgt_flat, emb3)

    logits = logits3.reshape(bt_pad, v_pad)
    if bt_pad != BT or v_pad != V:
        logits = logits[:BT, :V]

    if targets is None:
        return logits.reshape(B, T, V), None

    loss = jnp.sum(per_ex.reshape(bt_pad)[:BT]) / BT
    return logits, loss


# speedup vs baseline: 2.6107x; 2.6107x over previous
import functools

import jax
import jax.numpy as jnp
from jax import lax
from jax.experimental import pallas as pl
from jax.experimental.pallas import tpu as pltpu

_NEG = -1e30


def _round_up(x, m):
    return ((x + m - 1) // m) * m


def _row_lse_kernel(emb_ref, lse_ref):
    x = emb_ref[...]
    m = jnp.max(x, axis=1, keepdims=True)
    s = jnp.sum(jnp.exp(x - m), axis=1, keepdims=True)
    lse_ref[...] = m + jnp.log(s)


def _gather_loss_kernel(idx_sref, tgt_sref, emb_ref, lse_ref,
                        logits_ref, part_ref, *, tile, unroll, v_pad):
    base = pl.program_id(0) * tile
    lane = lax.broadcasted_iota(jnp.int32, (1, v_pad), 1)[0]

    def outer(o, carry):
        acc_sel, acc_lse = carry
        mi0 = o * unroll
        for u in range(unroll):
            mi = mi0 + u
            r = idx_sref[base + mi]
            t = tgt_sref[base + mi]
            row = emb_ref[r, 0, :]
            logits_ref[mi, 0, :] = row
            acc_sel = acc_sel + jnp.where(lane == t, row, 0.0)
            acc_lse = acc_lse + lse_ref[r, 0, :]
        return acc_sel, acc_lse

    acc_sel, acc_lse = lax.fori_loop(
        0, tile // unroll, outer,
        (jnp.zeros((v_pad,), jnp.float32), jnp.zeros((1,), jnp.float32)))

    part_ref[0, 0, :] = acc_lse - jnp.sum(acc_sel, keepdims=True)


def _pad_1d(tok, bt, bt_pad):
    tok = tok.reshape(bt).astype(jnp.int32)
    if bt_pad != bt:
        tok = jnp.concatenate([tok, jnp.zeros((bt_pad - bt,), jnp.int32)])
    return tok


def kernel(emb_table, idx, targets, *, tile=256, unroll=16):
    B, T = idx.shape
    V = emb_table.shape[0]
    BT = B * T

    v_pad = _round_up(V, 128)
    tile = min(tile, _round_up(BT, 8))
    bt_pad = _round_up(BT, tile)
    num_tiles = bt_pad // tile

    if v_pad == V:
        emb_pad = emb_table.astype(jnp.float32)
    else:
        emb_pad = jnp.full((v_pad, v_pad), _NEG, dtype=jnp.float32)
        emb_pad = emb_pad.at[:V, :V].set(emb_table.astype(jnp.float32))

    rtile = 256 if v_pad % 256 == 0 else 128
    lse_tab = pl.pallas_call(
        _row_lse_kernel,
        out_shape=jax.ShapeDtypeStruct((v_pad, 1), jnp.float32),
        grid=(v_pad // rtile,),
        in_specs=[pl.BlockSpec((rtile, v_pad), lambda i: (i, 0))],
        out_specs=pl.BlockSpec((rtile, 1), lambda i: (i, 0)),
        compiler_params=pltpu.CompilerParams(
            dimension_semantics=("parallel",),
        ),
        cost_estimate=pl.CostEstimate(
            flops=4 * v_pad * v_pad,
            transcendentals=v_pad * v_pad,
            bytes_accessed=v_pad * v_pad * 4 + v_pad * 4,
        ),
    )(emb_pad)

    emb3 = emb_pad.reshape(v_pad, 1, v_pad)
    lse3 = lse_tab.reshape(v_pad, 1, 1)
    idx_flat = _pad_1d(idx, BT, bt_pad)
    tgt_flat = _pad_1d(targets if targets is not None else idx, BT, bt_pad)

    body = functools.partial(_gather_loss_kernel, tile=tile, unroll=unroll,
                             v_pad=v_pad)

    logits3, partials = pl.pallas_call(
        body,
        out_shape=(
            jax.ShapeDtypeStruct((bt_pad, 1, v_pad), jnp.float32),
            jax.ShapeDtypeStruct((num_tiles, 1, 1), jnp.float32),
        ),
        grid=(num_tiles,),
        in_specs=[
            pl.BlockSpec(memory_space=pltpu.SMEM),
            pl.BlockSpec(memory_space=pltpu.SMEM),
            pl.BlockSpec((v_pad, 1, v_pad), lambda i: (0, 0, 0)),
            pl.BlockSpec((v_pad, 1, 1), lambda i: (0, 0, 0)),
        ],
        out_specs=(
            pl.BlockSpec((tile, 1, v_pad), lambda i: (i, 0, 0)),
            pl.BlockSpec((1, 1, 1), lambda i: (i, 0, 0)),
        ),
        compiler_params=pltpu.CompilerParams(
            dimension_semantics=("parallel",),
            vmem_limit_bytes=48 * 1024 * 1024,
        ),
        cost_estimate=pl.CostEstimate(
            flops=8 * bt_pad * v_pad,
            transcendentals=0,
            bytes_accessed=(v_pad * v_pad * 4 + bt_pad * v_pad * 4
                            + 2 * bt_pad * 4),
        ),
    )(idx_flat, tgt_flat, emb3, lse3)

    logits = logits3.reshape(bt_pad, v_pad)
    if bt_pad != BT or v_pad != V:
        logits = logits[:BT, :V]

    if targets is None:
        return logits.reshape(B, T, V), None

    loss_sum = jnp.sum(partials)
    if bt_pad != BT:
        n_pad = bt_pad - BT
        loss_sum = loss_sum - n_pad * (lse_tab[0, 0] - emb_pad[0, 0])
    return logits, loss_sum / BT


# speedup vs baseline: 4.3683x; 1.6733x over previous
import functools

import jax
import jax.numpy as jnp
from jax import lax
from jax.experimental import pallas as pl
from jax.experimental.pallas import tpu as pltpu

_NEG = -1e30


def _round_up(x, m):
    return ((x + m - 1) // m) * m


def _gather_ce_kernel(idx_sref, tgt_ref, emb_ref, logits_ref, part_ref,
                      *, tile, groups_per_trip, v_pad, bt):
    i = pl.program_id(0)
    base = i * tile

    def trip(o, carry):
        for g in range(groups_per_trip):
            row0 = o * groups_per_trip * 8 + g * 8
            rows = [emb_ref[idx_sref[base + row0 + k], 0, :]
                    for k in range(8)]
            x8 = jnp.stack(rows, axis=0)
            logits_ref[pl.ds(pl.multiple_of(row0, 8), 8), :] = x8
        return carry

    lax.fori_loop(0, tile // (8 * groups_per_trip), trip, 0)

    x = logits_ref[...]
    m = jnp.max(x, axis=-1, keepdims=True)
    s = jnp.sum(jnp.exp(x - m), axis=-1, keepdims=True)
    lane = lax.broadcasted_iota(jnp.int32, x.shape, 1)
    tgt_logit = jnp.sum(jnp.where(lane == tgt_ref[...], x, 0.0),
                        axis=-1, keepdims=True)
    per_ex = m + jnp.log(s) - tgt_logit
    row_id = base + lax.broadcasted_iota(jnp.int32, (tile, 1), 0)
    per_ex = jnp.where(row_id < bt, per_ex, 0.0)
    part_ref[0, 0, :] = jnp.sum(per_ex).reshape(1)


def _pad_1d(tok, bt, bt_pad):
    tok = tok.reshape(bt).astype(jnp.int32)
    if bt_pad != bt:
        tok = jnp.concatenate([tok, jnp.zeros((bt_pad - bt,), jnp.int32)])
    return tok


def kernel(emb_table, idx, targets, *, tile=256, groups_per_trip=4):
    B, T = idx.shape
    V = emb_table.shape[0]
    BT = B * T

    v_pad = _round_up(V, 128)
    tile = min(tile, _round_up(BT, 8))
    bt_pad = _round_up(BT, tile)
    num_tiles = bt_pad // tile

    if v_pad == V:
        emb_pad = emb_table.astype(jnp.float32)
    else:
        emb_pad = jnp.full((v_pad, v_pad), _NEG, dtype=jnp.float32)
        emb_pad = emb_pad.at[:V, :V].set(emb_table.astype(jnp.float32))

    emb3 = emb_pad.reshape(v_pad, 1, v_pad)
    idx_flat = _pad_1d(idx, BT, bt_pad)
    tgt_flat = _pad_1d(targets if targets is not None else idx, BT, bt_pad)
    tgt2 = tgt_flat.reshape(bt_pad, 1)

    body = functools.partial(_gather_ce_kernel, tile=tile,
                             groups_per_trip=groups_per_trip,
                             v_pad=v_pad, bt=BT)

    logits, partials = pl.pallas_call(
        body,
        out_shape=(
            jax.ShapeDtypeStruct((bt_pad, v_pad), jnp.float32),
            jax.ShapeDtypeStruct((num_tiles, 1, 1), jnp.float32),
        ),
        grid=(num_tiles,),
        in_specs=[
            pl.BlockSpec(memory_space=pltpu.SMEM),
            pl.BlockSpec((tile, 1), lambda i: (i, 0)),
            pl.BlockSpec((v_pad, 1, v_pad), lambda i: (0, 0, 0)),
        ],
        out_specs=(
            pl.BlockSpec((tile, v_pad), lambda i: (i, 0)),
            pl.BlockSpec((1, 1, 1), lambda i: (i, 0, 0)),
        ),
        compiler_params=pltpu.CompilerParams(
            dimension_semantics=("parallel",),
            vmem_limit_bytes=48 * 1024 * 1024,
        ),
        cost_estimate=pl.CostEstimate(
            flops=8 * bt_pad * v_pad,
            transcendentals=bt_pad * v_pad,
            bytes_accessed=(v_pad * v_pad * 4 + bt_pad * v_pad * 4
                            + 2 * bt_pad * 4),
        ),
    )(idx_flat, tgt2, emb3)

    if bt_pad != BT or v_pad != V:
        logits = logits[:BT, :V]

    if targets is None:
        return logits.reshape(B, T, V), None

    return logits, jnp.sum(partials) / BT


# speedup vs baseline: 4.5099x; 1.0324x over previous
import functools

import jax
import jax.numpy as jnp
from jax import lax
from jax.experimental import pallas as pl
from jax.experimental.pallas import tpu as pltpu

_NEG = -1e30


def _round_up(x, m):
    return ((x + m - 1) // m) * m


def _gather_ce_kernel(idx_sref, tgt_ref, emb_ref, logits_ref, part_ref,
                      *, tile, groups_per_trip, v_pad, bt):
    i = pl.program_id(0)
    base = i * tile

    def trip(o, carry):
        for g in range(groups_per_trip):
            row0 = o * groups_per_trip * 8 + g * 8
            rows = [emb_ref[idx_sref[base + row0 + k], 0, :]
                    for k in range(8)]
            x8 = jnp.stack(rows, axis=0)
            logits_ref[pl.ds(pl.multiple_of(row0, 8), 8), :] = x8
        return carry

    lax.fori_loop(0, tile // (8 * groups_per_trip), trip, 0)

    x = logits_ref[...]
    m = jnp.max(x, axis=-1, keepdims=True)
    s = jnp.sum(jnp.exp(x - m), axis=-1, keepdims=True)
    lane = lax.broadcasted_iota(jnp.int32, x.shape, 1)
    tgt_logit = jnp.sum(jnp.where(lane == tgt_ref[...], x, 0.0),
                        axis=-1, keepdims=True)
    per_ex = m + jnp.log(s) - tgt_logit
    row_id = base + lax.broadcasted_iota(jnp.int32, (tile, 1), 0)
    per_ex = jnp.where(row_id < bt, per_ex, 0.0)
    part_ref[0, 0, :] = jnp.sum(per_ex).reshape(1)


def _pad_1d(tok, bt, bt_pad):
    tok = tok.reshape(bt).astype(jnp.int32)
    if bt_pad != bt:
        tok = jnp.concatenate([tok, jnp.zeros((bt_pad - bt,), jnp.int32)])
    return tok


def kernel(emb_table, idx, targets, *, tile=512, groups_per_trip=4):
    B, T = idx.shape
    V = emb_table.shape[0]
    BT = B * T

    v_pad = _round_up(V, 128)
    tile = min(tile, _round_up(BT, 8))
    bt_pad = _round_up(BT, tile)
    num_tiles = bt_pad // tile

    if v_pad == V:
        emb_pad = emb_table.astype(jnp.float32)
    else:
        emb_pad = jnp.full((v_pad, v_pad), _NEG, dtype=jnp.float32)
        emb_pad = emb_pad.at[:V, :V].set(emb_table.astype(jnp.float32))

    emb3 = emb_pad.reshape(v_pad, 1, v_pad)
    idx_flat = _pad_1d(idx, BT, bt_pad)
    tgt_flat = _pad_1d(targets if targets is not None else idx, BT, bt_pad)
    tgt2 = tgt_flat.reshape(bt_pad, 1)

    body = functools.partial(_gather_ce_kernel, tile=tile,
                             groups_per_trip=groups_per_trip,
                             v_pad=v_pad, bt=BT)

    logits, partials = pl.pallas_call(
        body,
        out_shape=(
            jax.ShapeDtypeStruct((bt_pad, v_pad), jnp.float32),
            jax.ShapeDtypeStruct((num_tiles, 1, 1), jnp.float32),
        ),
        grid=(num_tiles,),
        in_specs=[
            pl.BlockSpec(memory_space=pltpu.SMEM),
            pl.BlockSpec((tile, 1), lambda i: (i, 0)),
            pl.BlockSpec((v_pad, 1, v_pad), lambda i: (0, 0, 0)),
        ],
        out_specs=(
            pl.BlockSpec((tile, v_pad), lambda i: (i, 0)),
            pl.BlockSpec((1, 1, 1), lambda i: (i, 0, 0)),
        ),
        compiler_params=pltpu.CompilerParams(
            dimension_semantics=("parallel",),
            vmem_limit_bytes=48 * 1024 * 1024,
        ),
        cost_estimate=pl.CostEstimate(
            flops=8 * bt_pad * v_pad,
            transcendentals=bt_pad * v_pad,
            bytes_accessed=(v_pad * v_pad * 4 + bt_pad * v_pad * 4
                            + 2 * bt_pad * 4),
        ),
    )(idx_flat, tgt2, emb3)

    if bt_pad != BT or v_pad != V:
        logits = logits[:BT, :V]

    if targets is None:
        return logits.reshape(B, T, V), None

    return logits, jnp.sum(partials) / BT


# speedup vs baseline: 4.5840x; 1.0164x over previous
import functools

import jax
import jax.numpy as jnp
from jax import lax
from jax.experimental import pallas as pl
from jax.experimental.pallas import tpu as pltpu

_NEG = -1e30


def _round_up(x, m):
    return ((x + m - 1) // m) * m


def _gather_ce_kernel(idx_sref, tgt_ref, emb_ref, logits_ref, part_ref,
                      *, tile, groups_per_trip, v_pad, bt):
    i = pl.program_id(0)
    base = i * tile

    def trip(o, carry):
        for g in range(groups_per_trip):
            row0 = o * groups_per_trip * 8 + g * 8
            rows = [emb_ref[idx_sref[base + row0 + k], 0, :]
                    for k in range(8)]
            x8 = jnp.stack(rows, axis=0)
            logits_ref[pl.ds(pl.multiple_of(row0, 8), 8), :] = x8
        return carry

    lax.fori_loop(0, tile // (8 * groups_per_trip), trip, 0)

    x = logits_ref[...]
    m = jnp.max(x, axis=-1, keepdims=True)
    s = jnp.sum(jnp.exp(x - m), axis=-1, keepdims=True)
    lane = lax.broadcasted_iota(jnp.int32, x.shape, 1)
    tgt_logit = jnp.sum(jnp.where(lane == tgt_ref[...], x, 0.0),
                        axis=-1, keepdims=True)
    per_ex = m + jnp.log(s) - tgt_logit
    row_id = base + lax.broadcasted_iota(jnp.int32, (tile, 1), 0)
    per_ex = jnp.where(row_id < bt, per_ex, 0.0)
    part_ref[0, 0, :] = jnp.sum(per_ex).reshape(1)


def _pad_1d(tok, bt, bt_pad):
    tok = tok.reshape(bt).astype(jnp.int32)
    if bt_pad != bt:
        tok = jnp.concatenate([tok, jnp.zeros((bt_pad - bt,), jnp.int32)])
    return tok


def kernel(emb_table, idx, targets, *, tile=1024, groups_per_trip=4):
    B, T = idx.shape
    V = emb_table.shape[0]
    BT = B * T

    v_pad = _round_up(V, 128)
    tile = min(tile, _round_up(BT, 8))
    bt_pad = _round_up(BT, tile)
    num_tiles = bt_pad // tile

    if v_pad == V:
        emb_pad = emb_table.astype(jnp.float32)
    else:
        emb_pad = jnp.full((v_pad, v_pad), _NEG, dtype=jnp.float32)
        emb_pad = emb_pad.at[:V, :V].set(emb_table.astype(jnp.float32))

    emb3 = emb_pad.reshape(v_pad, 1, v_pad)
    idx_flat = _pad_1d(idx, BT, bt_pad)
    tgt_flat = _pad_1d(targets if targets is not None else idx, BT, bt_pad)
    tgt2 = tgt_flat.reshape(bt_pad, 1)

    body = functools.partial(_gather_ce_kernel, tile=tile,
                             groups_per_trip=groups_per_trip,
                             v_pad=v_pad, bt=BT)

    logits, partials = pl.pallas_call(
        body,
        out_shape=(
            jax.ShapeDtypeStruct((bt_pad, v_pad), jnp.float32),
            jax.ShapeDtypeStruct((num_tiles, 1, 1), jnp.float32),
        ),
        grid=(num_tiles,),
        in_specs=[
            pl.BlockSpec(memory_space=pltpu.SMEM),
            pl.BlockSpec((tile, 1), lambda i: (i, 0)),
            pl.BlockSpec((v_pad, 1, v_pad), lambda i: (0, 0, 0)),
        ],
        out_specs=(
            pl.BlockSpec((tile, v_pad), lambda i: (i, 0)),
            pl.BlockSpec((1, 1, 1), lambda i: (i, 0, 0)),
        ),
        compiler_params=pltpu.CompilerParams(
            dimension_semantics=("parallel",),
            vmem_limit_bytes=48 * 1024 * 1024,
        ),
        cost_estimate=pl.CostEstimate(
            flops=8 * bt_pad * v_pad,
            transcendentals=bt_pad * v_pad,
            bytes_accessed=(v_pad * v_pad * 4 + bt_pad * v_pad * 4
                            + 2 * bt_pad * 4),
        ),
    )(idx_flat, tgt2, emb3)

    if bt_pad != BT or v_pad != V:
        logits = logits[:BT, :V]

    if targets is None:
        return logits.reshape(B, T, V), None

    return logits, jnp.sum(partials) / BT
